# baseline (speedup 1.0000x reference)
import jax
import jax.numpy as jnp
from jax.experimental import pallas as pl
from jax.experimental.pallas import tpu as pltpu

_B = 1024
_F = 4 * 64 * 64
_TPAD = 1024
_BB = 64


def _body(t_ref, a_ref, b_ref, x_ref, n_ref, o_ref):
    t_col = t_ref[...]
    lane = jax.lax.broadcasted_iota(jnp.int32, (_BB, _TPAD), 1)
    onehot = t_col == lane
    av = jnp.sum(jnp.where(onehot, a_ref[...], 0.0), axis=1, keepdims=True)
    bv = jnp.sum(jnp.where(onehot, b_ref[...], 0.0), axis=1, keepdims=True)
    o_ref[...] = av * x_ref[...] + bv * n_ref[...]


def kernel(x_0, timestep, noise, a, b):
    x2 = x_0.reshape(_B, _F)
    n2 = noise.reshape(_B, _F)
    t2 = timestep.reshape(_B, 1).astype(jnp.int32)
    ap = jnp.pad(a, (0, _TPAD - a.shape[0])).reshape(1, _TPAD)
    bp = jnp.pad(b, (0, _TPAD - b.shape[0])).reshape(1, _TPAD)

    grid = (_B // _BB,)
    out = pl.pallas_call(
        _body,
        grid=grid,
        in_specs=[
            pl.BlockSpec((_BB, 1), lambda i: (i, 0)),
            pl.BlockSpec((1, _TPAD), lambda i: (0, 0)),
            pl.BlockSpec((1, _TPAD), lambda i: (0, 0)),
            pl.BlockSpec((_BB, _F), lambda i: (i, 0)),
            pl.BlockSpec((_BB, _F), lambda i: (i, 0)),
        ],
        out_specs=pl.BlockSpec((_BB, _F), lambda i: (i, 0)),
        out_shape=jax.ShapeDtypeStruct((_B, _F), jnp.float32),
        compiler_params=pltpu.CompilerParams(
            dimension_semantics=("arbitrary",),
        ),
    )(t2, ap, bp, x2, n2)
    return out.reshape(x_0.shape)
